# baseline (speedup 1.0000x reference)
import functools

import jax
import jax.numpy as jnp
from jax import lax
from jax.experimental import pallas as pl
from jax.experimental.pallas import tpu as pltpu
from jax.experimental.pallas import tpu_sc as plsc

_NC = 2
_NS = 16
_NW = _NC * _NS


def _make_gather(N, S, V, D, R, NB):
    assert N % (_NW * R * NB) == 0
    rows_per_w = N // _NW
    n_chunks = rows_per_w // R
    n_groups = n_chunks // NB
    mesh = plsc.VectorSubcoreMesh(core_axis_name="c", subcore_axis_name="s")

    scratch = (
        [pltpu.VMEM((rows_per_w, S), jnp.int32)]
        + [pltpu.VMEM((R, S, D), jnp.float32) for _ in range(NB)]
        + [pltpu.SemaphoreType.DMA for _ in range(2 * NB)]
    )

    @functools.partial(
        pl.kernel,
        mesh=mesh,
        out_type=jax.ShapeDtypeStruct((N, S, D), jnp.float32),
        scratch_types=scratch,
        compiler_params=pltpu.CompilerParams(use_tc_tiling_on_sc=False),
    )
    def gather_kernel(idx_hbm, table_hbm, out_hbm, idx_v, *rest):
        bufs = rest[:NB]
        gsems = rest[NB:2 * NB]
        osems = rest[2 * NB:]
        wid = lax.axis_index("s") * _NC + lax.axis_index("c")
        row_base = wid * rows_per_w
        pltpu.sync_copy(idx_hbm.at[pl.ds(row_base, rows_per_w)], idx_v)

        def gather(j, b):
            for r in range(R):
                pltpu.async_copy(
                    table_hbm.at[idx_v.at[j * R + r]], bufs[b].at[r],
                    gsems[b])

        def wait_gather(j, b):
            for r in range(R):
                pltpu.make_async_copy(
                    table_hbm.at[idx_v.at[j * R + r]], bufs[b].at[r],
                    gsems[b]).wait()

        def write(j, b):
            pltpu.async_copy(
                bufs[b], out_hbm.at[pl.ds(row_base + j * R, R)], osems[b])

        def wait_write(j, b):
            pltpu.make_async_copy(
                bufs[b], out_hbm.at[pl.ds(row_base + j * R, R)],
                osems[b]).wait()

        for b in range(NB):
            gather(b, b)

        def body(g, carry):
            for b in range(NB):
                wait_gather(g * NB + b, b)
                write(g * NB + b, b)

            @pl.when(g + 1 < n_groups)
            def _():
                for b in range(NB):
                    wait_write(g * NB + b, b)
                    gather((g + 1) * NB + b, b)

            return carry

        lax.fori_loop(0, n_groups, body, 0)
        for b in range(NB):
            wait_write((n_groups - 1) * NB + b, b)

    return gather_kernel


def kernel(input, table):
    N, S = input.shape
    V, D = table.shape
    return _make_gather(N, S, V, D, 16, 4)(input, table)
